# Initial kernel scaffold; baseline (speedup 1.0000x reference)
import jax
import jax.numpy as jnp
from jax import lax
from jax.experimental import pallas as pl
from jax.experimental.pallas import tpu as pltpu
from jax.experimental.pallas import tpu_sc as plsc

N = 100000
F = 200000
NTILE = 16
NCORE = 2
NW = NCORE * NTILE
NP = 100096
RV = NP // NTILE
CH = 128
NCHUNK = 49
FW = NCHUNK * CH
FP = FW * NW
NFL = NP * 4 // 128


def _sub_body(a_ref, b_ref, o_ref):
    o_ref[...] = a_ref[...] - b_ref[...]


def _sc_body(d4_hbm, z_hbm, f0_hbm, f1_hbm, f2_hbm, part_hbm,
             d4_s, acc_s, idx0, idx1, idx2, rows0, rows1, rows2):
    cid = lax.axis_index("c")
    sid = lax.axis_index("s")
    wid = cid * NTILE + sid
    base_v = sid * RV
    vsl = pl.ds(base_v, RV)
    pltpu.sync_copy(d4_hbm.at[vsl], d4_s.at[vsl])
    pltpu.sync_copy(z_hbm, acc_s.at[vsl])
    pltpu.sync_copy(f0_hbm.at[wid], idx0)
    pltpu.sync_copy(f1_hbm.at[wid], idx1)
    pltpu.sync_copy(f2_hbm.at[wid], idx2)
    plsc.subcore_barrier()

    @pl.loop(0, NCHUNK)
    def _chunk(j):
        i0 = idx0.at[j]
        i1 = idx1.at[j]
        i2 = idx2.at[j]
        pltpu.sync_copy(d4_s.at[i0], rows0)
        pltpu.sync_copy(d4_s.at[i1], rows1)
        pltpu.sync_copy(d4_s.at[i2], rows2)
        pltpu.sync_copy(rows1, acc_s.at[i0], add=True)
        pltpu.sync_copy(rows2, acc_s.at[i0], add=True)
        pltpu.sync_copy(rows0, acc_s.at[i1], add=True)
        pltpu.sync_copy(rows2, acc_s.at[i1], add=True)
        pltpu.sync_copy(rows0, acc_s.at[i2], add=True)
        pltpu.sync_copy(rows1, acc_s.at[i2], add=True)

    plsc.subcore_barrier()
    pltpu.sync_copy(acc_s.at[vsl], part_hbm.at[cid, vsl])


def _loss_body(part_ref, v1_ref, v2_ref, o_ref):
    a = part_ref[0] + part_ref[1]
    d = v1_ref[...] - v2_ref[...]
    lane = lax.broadcasted_iota(jnp.int32, (NFL, 128), 1)
    is3 = (lane & 3) == 3
    c3 = jnp.where(is3, a, 0.0)
    cb = (c3 + jnp.roll(c3, -1, 1) + jnp.roll(c3, -2, 1)
          + jnp.roll(c3, -3, 1))
    deg = jnp.maximum(cb, 1.0)
    lap = a / deg - d
    r = jnp.where(is3, 0.0, jnp.abs(lap))
    o_ref[0, 0] = jnp.sum(r) * (1.0 / (N * 3.0))


def kernel(vert1, vert2, face):
    f32 = jnp.float32
    v1p = jnp.pad(
        jnp.concatenate([vert1, jnp.ones((N, 1), f32)], axis=1),
        ((0, NP - N), (0, 0)))
    v2p = jnp.pad(
        jnp.concatenate([vert2, jnp.zeros((N, 1), f32)], axis=1),
        ((0, NP - N), (0, 0)))
    v1f = v1p.reshape(NFL, 128)
    v2f = v2p.reshape(NFL, 128)

    d4f = pl.pallas_call(
        _sub_body,
        out_shape=jax.ShapeDtypeStruct((NFL, 128), f32),
    )(v1f, v2f)
    d4 = d4f.reshape(NP, 4)

    fpad = jnp.full((FP - F,), N, jnp.int32)
    f0 = jnp.concatenate([face[:, 0], fpad]).reshape(NW, NCHUNK, CH)
    f1 = jnp.concatenate([face[:, 1], fpad]).reshape(NW, NCHUNK, CH)
    f2 = jnp.concatenate([face[:, 2], fpad]).reshape(NW, NCHUNK, CH)
    z = jnp.zeros((RV, 4), f32)

    sc = pl.kernel(
        _sc_body,
        out_type=jax.ShapeDtypeStruct((NCORE, NP, 4), f32),
        mesh=plsc.VectorSubcoreMesh(core_axis_name="c", subcore_axis_name="s",
                                    num_cores=NCORE, num_subcores=NTILE),
        scratch_types=[
            pltpu.VMEM_SHARED((NP, 4), f32),
            pltpu.VMEM_SHARED((NP, 4), f32),
            pltpu.VMEM((NCHUNK, CH), jnp.int32),
            pltpu.VMEM((NCHUNK, CH), jnp.int32),
            pltpu.VMEM((NCHUNK, CH), jnp.int32),
            pltpu.VMEM((CH, 4), f32),
            pltpu.VMEM((CH, 4), f32),
            pltpu.VMEM((CH, 4), f32),
        ],
    )
    part = sc(d4, z, f0, f1, f2)

    out = pl.pallas_call(
        _loss_body,
        out_shape=jax.ShapeDtypeStruct((1, 1), f32),
        out_specs=pl.BlockSpec(memory_space=pltpu.SMEM),
    )(part.reshape(NCORE, NFL, 128), v1f, v2f)
    return out[0, 0]


# speedup vs baseline: 50.0251x; 50.0251x over previous
import jax
import jax.numpy as jnp
from jax import lax
from jax.experimental import pallas as pl
from jax.experimental.pallas import tpu as pltpu
from jax.experimental.pallas import tpu_sc as plsc

N = 100000
F = 200000
NP = 100352
SLAB = NP // 4
NG = 8
IB = 512
NB = 50
FG = NB * IB
FP = NG * FG
NROW = NP // 128


def _sub_body(a_ref, b_ref, o_ref):
    o_ref[...] = a_ref[...] - b_ref[...]


def _sc_body(d4_hbm, fidx_hbm, part_hbm, dc, acc, idxa, idxb, sema, semb):
    cid = lax.axis_index("c")
    sid = lax.axis_index("s")
    gw = cid * 4 + (sid >> 2)
    p = sid & 3

    pltpu.sync_copy(d4_hbm.at[p], dc)

    ibufs = (idxa, idxb)
    sems = (sema, semb)

    @pl.loop(0, 4)
    def _slab(s):
        lo = s * SLAB

        @pl.loop(0, SLAB // 16)
        def _zero(i):
            acc[pl.ds(i * 16, 16)] = jnp.zeros((16,), jnp.float32)

        pltpu.async_copy(fidx_hbm.at[gw, 0], idxa, sema)

        @pl.loop(0, NB)
        def _block(b):
            cur = b % 2
            for t in range(2):
                @pl.when(cur == t)
                def _():
                    pltpu.make_async_copy(
                        fidx_hbm.at[gw, b], ibufs[t], sems[t]).wait()

            @pl.when(b + 1 < NB)
            def _():
                nxt = (b + 1) % 2
                for t in range(2):
                    @pl.when(nxt == t)
                    def _():
                        pltpu.async_copy(
                            fidx_hbm.at[gw, b + 1], ibufs[t], sems[t])

            for t in range(2):
                @pl.when(cur == t)
                def _():
                    ibuf = ibufs[t]

                    @pl.loop(0, IB // 16)
                    def _vec(k):
                        ksl = pl.ds(k * 16, 16)
                        i0 = ibuf[0, ksl]
                        i1 = ibuf[1, ksl]
                        i2 = ibuf[2, ksl]
                        a0 = plsc.load_gather(dc, [i0])
                        a1 = plsc.load_gather(dc, [i1])
                        a2 = plsc.load_gather(dc, [i2])
                        s3 = a0 + a1 + a2
                        j0 = i0 - lo
                        j1 = i1 - lo
                        j2 = i2 - lo
                        m0 = (j0 >= 0) & (j0 < SLAB)
                        m1 = (j1 >= 0) & (j1 < SLAB)
                        m2 = (j2 >= 0) & (j2 < SLAB)
                        plsc.addupdate_scatter(acc, [j0], s3 - a0, mask=m0)
                        plsc.addupdate_scatter(acc, [j1], s3 - a1, mask=m1)
                        plsc.addupdate_scatter(acc, [j2], s3 - a2, mask=m2)

        pltpu.sync_copy(acc, part_hbm.at[gw, p, pl.ds(lo, SLAB)])


def _loss_body(part_ref, d_ref, o_ref):
    a = part_ref[0]
    for g in range(1, NG):
        a = a + part_ref[g]
    deg = jnp.maximum(a[3], 1.0)
    lap = a[0:3] / deg[None] - d_ref[0:3]
    o_ref[0, 0] = jnp.sum(jnp.abs(lap)) * (1.0 / (N * 3.0))


def kernel(vert1, vert2, face):
    f32 = jnp.float32
    v1t = jnp.concatenate(
        [jnp.pad(vert1.T, ((0, 0), (0, NP - N))), jnp.ones((1, NP), f32)])
    v2t = jnp.concatenate(
        [jnp.pad(vert2.T, ((0, 0), (0, NP - N))), jnp.zeros((1, NP), f32)])

    d4f = pl.pallas_call(
        _sub_body,
        out_shape=jax.ShapeDtypeStruct((4, NROW, 128), f32),
    )(v1t.reshape(4, NROW, 128), v2t.reshape(4, NROW, 128))
    d4t = d4f.reshape(4, NP)

    ft = jnp.pad(face.T, ((0, 0), (0, FP - F)), constant_values=N)
    fidx = ft.reshape(3, NG, NB, IB).transpose(1, 2, 0, 3)

    mesh = plsc.VectorSubcoreMesh(core_axis_name="c", subcore_axis_name="s",
                                  num_cores=2, num_subcores=16)
    sc = pl.kernel(
        _sc_body,
        out_type=jax.ShapeDtypeStruct((NG, 4, NP), f32),
        mesh=mesh,
        compiler_params=pltpu.CompilerParams(use_tc_tiling_on_sc=False,
                                             needs_layout_passes=False),
        scratch_types=[
            pltpu.VMEM((NP,), f32),
            pltpu.VMEM((SLAB,), f32),
            pltpu.VMEM((3, IB), jnp.int32),
            pltpu.VMEM((3, IB), jnp.int32),
            pltpu.SemaphoreType.DMA,
            pltpu.SemaphoreType.DMA,
        ],
    )
    part = sc(d4t, fidx)

    out = pl.pallas_call(
        _loss_body,
        out_shape=jax.ShapeDtypeStruct((1, 1), f32),
        out_specs=pl.BlockSpec(memory_space=pltpu.SMEM),
    )(part.reshape(NG, 4, NROW, 128), d4f)
    return out[0, 0]
